# baseline (speedup 1.0000x reference)
import functools

import jax
import jax.numpy as jnp
from jax import lax
from jax.experimental import pallas as pl
from jax.experimental.pallas import tpu as pltpu
from jax.experimental.pallas import tpu_sc as plsc

L = 16
NBUF = 4


@functools.lru_cache(maxsize=None)
def _build(B, NCAT, NNUM, VROWS, D):
    info = plsc.get_sparse_core_info()
    NC, NS = info.num_cores, info.num_subcores
    NW = NC * NS
    NTOK = 1 + NCAT + NNUM
    KD = D // L
    CB = 4
    RW = B // NW
    NCHUNK = RW // CB
    assert B % (NW * CB) == 0 and D % L == 0 and NCHUNK % NBUF == 0

    mesh = plsc.VectorSubcoreMesh(core_axis_name="c", subcore_axis_name="s")

    @functools.partial(
        pl.kernel,
        out_type=jax.ShapeDtypeStruct((B, NTOK, D), jnp.float32),
        mesh=mesh,
        compiler_params=pltpu.CompilerParams(use_tc_tiling_on_sc=False),
        scratch_types=(
            [pltpu.VMEM((NCAT, CB), jnp.int32)] * NBUF
            + [pltpu.VMEM((CB, L), jnp.float32)] * NBUF
            + [pltpu.VMEM((NCAT, CB, D), jnp.float32)] * NBUF
            + [pltpu.VMEM((CB, NTOK, D), jnp.float32)] * NBUF
            + [pltpu.VMEM((NCAT, D), jnp.float32),
               pltpu.VMEM((NNUM, D), jnp.float32),
               pltpu.VMEM((NNUM, D), jnp.float32)]
            + [pltpu.VMEM((D,), jnp.float32)]
            + [pltpu.SemaphoreType.DMA] * NBUF
            + [pltpu.SemaphoreType.DMA] * NBUF
            + [pltpu.SemaphoreType.DMA] * NBUF
        ),
    )
    def tokenize(idxt_hbm, xnum_hbm, tab3_hbm, pos_hbm, w_hbm, add_hbm,
                 cls_hbm, out_hbm, *refs):
        idx_v = refs[0:NBUF]
        xnum_v = refs[NBUF:2 * NBUF]
        gat_v = refs[2 * NBUF:3 * NBUF]
        out_v = refs[3 * NBUF:4 * NBUF]
        pos_v, w_v, add_v, cls_v = refs[4 * NBUF:4 * NBUF + 4]
        isem = refs[4 * NBUF + 4:5 * NBUF + 4]
        gsem = refs[5 * NBUF + 4:6 * NBUF + 4]
        wsem = refs[6 * NBUF + 4:7 * NBUF + 4]

        wid = lax.axis_index("s") * NC + lax.axis_index("c")
        base = wid * RW
        pltpu.sync_copy(pos_hbm, pos_v)
        pltpu.sync_copy(w_hbm, w_v)
        pltpu.sync_copy(add_hbm, add_v)
        pltpu.sync_copy(cls_hbm, cls_v)

        def load_inputs(g, b):
            b0 = base + g * CB
            pltpu.async_copy(idxt_hbm.at[wid, g], idx_v[b], isem[b])
            pltpu.async_copy(xnum_hbm.at[pl.ds(b0, CB)], xnum_v[b], isem[b])

        def wait_inputs(g, b):
            b0 = base + g * CB
            pltpu.make_async_copy(idxt_hbm.at[wid, g], idx_v[b],
                                  isem[b]).wait()
            pltpu.make_async_copy(xnum_hbm.at[pl.ds(b0, CB)], xnum_v[b],
                                  isem[b]).wait()

        def fire_gathers(b, lo, hi):
            return [
                pltpu.async_copy(tab3_hbm.at[i].at[idx_v[b].at[i]],
                                 gat_v[b].at[i], gsem[b])
                for i in range(lo, hi)
            ]

        def fill_and_drain(b, descs):
            def fill_row(r, c):
                for k in range(KD):
                    out_v[b][r, 0, pl.ds(k * L, L)] = cls_v[pl.ds(k * L, L)]
                xv = xnum_v[b][r, :]
                for j in range(NNUM):
                    x = xv[j]
                    for k in range(KD):
                        out_v[b][r, 1 + NCAT + j, pl.ds(k * L, L)] = (
                            x * w_v[j, pl.ds(k * L, L)]
                            + add_v[j, pl.ds(k * L, L)])
                return c

            lax.fori_loop(0, CB, fill_row, 0)
            for d in descs:
                d.wait()

            def add_pos(r, c):
                for i in range(NCAT):
                    for k in range(KD):
                        sl = pl.ds(k * L, L)
                        out_v[b][r, 1 + i, sl] = (
                            gat_v[b][i, r, sl] + pos_v[i, sl])
                return c

            lax.fori_loop(0, CB, add_pos, 0)

        def issue_writeback(g, b):
            b0 = base + g * CB
            pltpu.async_copy(out_v[b], out_hbm.at[pl.ds(b0, CB)], wsem[b])

        def wait_writeback(g, b):
            b0 = base + g * CB
            pltpu.make_async_copy(out_v[b], out_hbm.at[pl.ds(b0, CB)],
                                  wsem[b]).wait()

        for b in range(NBUF - 1):
            load_inputs(b, b)

        def ring(h, carry):
            g0 = h * NBUF
            for b in range(NBUF):
                g = g0 + b
                @pl.when(g >= NBUF)
                def _(b=b, g=g):
                    wait_writeback(g - NBUF, b)

                wait_inputs(g, b)
                descs = fire_gathers(b, 0, NCAT // 2)
                @pl.when(g + NBUF - 1 < NCHUNK)
                def _(b=b, g=g):
                    load_inputs(g + NBUF - 1, (b + NBUF - 1) % NBUF)

                descs += fire_gathers(b, NCAT // 2, NCAT)
                fill_and_drain(b, descs)
                issue_writeback(g, b)
            return carry

        lax.fori_loop(0, NCHUNK // NBUF, ring, 0)
        for b in range(NBUF):
            wait_writeback(NCHUNK - NBUF + b, b)

    return tokenize


def kernel(x_cat, x_num, cat_tables, num_w, num_b, feature_pos, cls):
    B, NCAT = x_cat.shape
    NNUM = x_num.shape[1]
    VROWS, D = cat_tables.shape[1], cat_tables.shape[2]
    NW = 32
    CB = 4
    idxt = (x_cat.astype(jnp.int32)
            .reshape(NW, B // (NW * CB), CB, NCAT)
            .transpose(0, 1, 3, 2))
    pos_cat = feature_pos[1:1 + NCAT]
    num_add = num_b + feature_pos[1 + NCAT:]
    cls_row = cls.reshape(D) + feature_pos[0]
    xnum_pad = jnp.zeros((B, L), dtype=jnp.float32).at[:, :NNUM].set(
        x_num.astype(jnp.float32))
    fn = _build(B, NCAT, NNUM, VROWS, D)
    return fn(idxt, xnum_pad, cat_tables, pos_cat, num_w, num_add, cls_row)


# speedup vs baseline: 1.4155x; 1.4155x over previous
import functools

import jax
import jax.numpy as jnp
from jax import lax
from jax.experimental import pallas as pl
from jax.experimental.pallas import tpu as pltpu
from jax.experimental.pallas import tpu_sc as plsc

L = 16
NBUF = 4
CB = 4


@functools.lru_cache(maxsize=None)
def _build(B, NCAT, NNUM, VROWS, D):
    info = plsc.get_sparse_core_info()
    NC, NS = info.num_cores, info.num_subcores
    NW = NC * NS
    NTOK = 1 + NCAT + NNUM
    KD = D // L
    RW = B // NW
    NCHUNK = RW // CB
    assert B % (NW * CB) == 0 and D % L == 0 and NCHUNK % NBUF == 0
    CW = NCAT + NNUM
    CA = NCAT + 2 * NNUM
    CROWS = -(-(CA + 1) // 8) * 8

    mesh = plsc.VectorSubcoreMesh(core_axis_name="c", subcore_axis_name="s")

    @functools.partial(
        pl.kernel,
        out_type=jax.ShapeDtypeStruct((B, NTOK, D), jnp.float32),
        mesh=mesh,
        scratch_types=(
            [pltpu.VMEM((8, 128), jnp.int32)] * NBUF
            + [pltpu.VMEM((8, 128), jnp.float32)] * NBUF
            + [pltpu.VMEM((CB, NTOK, D), jnp.float32)] * NBUF
            + [pltpu.VMEM((CROWS, 128), jnp.float32)]
            + [pltpu.SemaphoreType.DMA] * NBUF
            + [pltpu.SemaphoreType.DMA] * NBUF
            + [pltpu.SemaphoreType.DMA] * NBUF
        ),
    )
    def tokenize(blk_hbm, xblk_hbm, tab3_hbm, const_hbm, out_hbm, *refs):
        idx_v = refs[0:NBUF]
        xnum_v = refs[NBUF:2 * NBUF]
        out_v = refs[2 * NBUF:3 * NBUF]
        const_v = refs[3 * NBUF]
        isem = refs[3 * NBUF + 1:4 * NBUF + 1]
        gsem = refs[4 * NBUF + 1:5 * NBUF + 1]
        wsem = refs[5 * NBUF + 1:6 * NBUF + 1]

        wid = lax.axis_index("s") * NC + lax.axis_index("c")
        base = wid * RW
        pltpu.sync_copy(const_hbm, const_v)

        def load_inputs(g, b):
            pltpu.async_copy(blk_hbm.at[wid, g], idx_v[b], isem[b])
            pltpu.async_copy(xblk_hbm.at[wid, g], xnum_v[b], isem[b])

        def wait_inputs(g, b):
            pltpu.make_async_copy(blk_hbm.at[wid, g], idx_v[b],
                                  isem[b]).wait()
            pltpu.make_async_copy(xblk_hbm.at[wid, g], xnum_v[b],
                                  isem[b]).wait()

        def gather_and_fill(b):
            def row(r, c):
                v0 = idx_v[b][r, pl.ds(0, L)]
                v1 = idx_v[b][r, pl.ds(L, L)]
                for i in range(NCAT):
                    rowid = v0[i] if i < L else v1[i - L]
                    pltpu.async_copy(
                        tab3_hbm.at[i, pl.ds(rowid, 1), :],
                        out_v[b].at[pl.ds(r, 1), 1 + i, :], gsem[b])
                xv = xnum_v[b][r, pl.ds(0, L)]
                for k in range(KD):
                    sl = pl.ds(k * L, L)
                    out_v[b][r, 0, sl] = const_v[CA, sl]
                for j in range(NNUM):
                    x = xv[j]
                    for k in range(KD):
                        sl = pl.ds(k * L, L)
                        out_v[b][r, 1 + NCAT + j, sl] = (
                            x * const_v[NCAT + j, sl] + const_v[CW + j, sl])
                return c

            lax.fori_loop(0, CB, row, 0)

        def drain_and_addpos(b):
            def wrow(r, c):
                for i in range(NCAT):
                    pltpu.make_async_copy(
                        tab3_hbm.at[i, pl.ds(0, 1), :],
                        out_v[b].at[pl.ds(r, 1), 1 + i, :], gsem[b]).wait()
                return c

            lax.fori_loop(0, CB, wrow, 0)

            def add_pos(r, c):
                for i in range(NCAT):
                    for k in range(KD):
                        sl = pl.ds(k * L, L)
                        out_v[b][r, 1 + i, sl] = (
                            out_v[b][r, 1 + i, sl] + const_v[i, sl])
                return c

            lax.fori_loop(0, CB, add_pos, 0)

        def issue_writeback(g, b):
            b0 = base + g * CB
            pltpu.async_copy(out_v[b], out_hbm.at[pl.ds(b0, CB)], wsem[b])

        def wait_writeback(g, b):
            b0 = base + g * CB
            pltpu.make_async_copy(out_v[b], out_hbm.at[pl.ds(b0, CB)],
                                  wsem[b]).wait()

        for b in range(NBUF - 1):
            load_inputs(b, b)

        def ring(h, carry):
            g0 = h * NBUF
            for b in range(NBUF):
                g = g0 + b
                @pl.when(g >= NBUF)
                def _(b=b, g=g):
                    wait_writeback(g - NBUF, b)

                wait_inputs(g, b)
                gather_and_fill(b)
                @pl.when(g + NBUF - 1 < NCHUNK)
                def _(b=b, g=g):
                    load_inputs(g + NBUF - 1, (b + NBUF - 1) % NBUF)

                drain_and_addpos(b)
                issue_writeback(g, b)
            return carry

        lax.fori_loop(0, NCHUNK // NBUF, ring, 0)
        for b in range(NBUF):
            wait_writeback(NCHUNK - NBUF + b, b)

    return tokenize


def kernel(x_cat, x_num, cat_tables, num_w, num_b, feature_pos, cls):
    B, NCAT = x_cat.shape
    NNUM = x_num.shape[1]
    VROWS, D = cat_tables.shape[1], cat_tables.shape[2]
    NW = 32
    NCHUNK = B // (NW * CB)
    CW = NCAT + NNUM
    CA = NCAT + 2 * NNUM
    CROWS = -(-(CA + 1) // 8) * 8
    xi = x_cat.astype(jnp.int32).reshape(NW, NCHUNK, CB, NCAT)
    xf = x_num.astype(jnp.float32).reshape(NW, NCHUNK, CB, NNUM)
    blk = (jnp.zeros((NW, NCHUNK, 8, 128), jnp.int32)
           .at[:, :, :CB, :NCAT].set(xi))
    xblk = (jnp.zeros((NW, NCHUNK, 8, 128), jnp.float32)
            .at[:, :, :CB, :NNUM].set(xf))
    cpad = (jnp.zeros((CROWS, 128), jnp.float32)
            .at[:NCAT, :D].set(feature_pos[1:1 + NCAT])
            .at[NCAT:CW, :D].set(num_w)
            .at[CW:CA, :D].set(num_b + feature_pos[1 + NCAT:])
            .at[CA, :D].set(cls.reshape(D) + feature_pos[0]))
    fn = _build(B, NCAT, NNUM, VROWS, D)
    return fn(blk, xblk, cat_tables, cpad)
